# baseline (speedup 1.0000x reference)
import jax
import jax.numpy as jnp
from jax.experimental import pallas as pl

TOPN = 1000
MIN_SCORE = 0.05
NMS_THR = 0.5
MAX_OBJ = 100

L = 3
B = 4
N = 17328


def _decode_kernel(h_ref, obj_ref, r0_ref, r1_ref, r2_ref, r3_ref,
                   a0_ref, a1_ref, a2_ref, a3_ref, a4_ref,
                   s_ref, c_ref, x1_ref, y1_ref, x2_ref, y2_ref):
    h = h_ref[0]
    cls = h[:, 5:85]
    m = jnp.max(cls, axis=1)
    iota = jax.lax.broadcasted_iota(jnp.int32, cls.shape, 1)
    c = jnp.min(jnp.where(cls == m[:, None], iota, 10_000), axis=1)
    obj = obj_ref[0, 0, 0]
    s = m * obj
    a0 = a0_ref[0, 0, 0]; a1 = a1_ref[0, 0, 0]; a2 = a2_ref[0, 0, 0]
    a3 = a3_ref[0, 0, 0]; a4 = a4_ref[0, 0, 0]
    cx = (r0_ref[0, 0, 0] + a0) * a4
    cy = (r1_ref[0, 0, 0] + a1) * a4
    w = r2_ref[0, 0, 0] * a2 * a4
    hh = r3_ref[0, 0, 0] * a3 * a4
    s_ref[0, 0, 0] = s
    c_ref[0, 0, 0] = c.astype(jnp.float32)
    x1_ref[0, 0, 0] = (cx - w * 0.5).astype(jnp.int32).astype(jnp.float32)
    y1_ref[0, 0, 0] = (cy - hh * 0.5).astype(jnp.int32).astype(jnp.float32)
    x2_ref[0, 0, 0] = (cx + w * 0.5).astype(jnp.int32).astype(jnp.float32)
    y2_ref[0, 0, 0] = (cy + hh * 0.5).astype(jnp.int32).astype(jnp.float32)


def _decode(obj_reg_cls_heads, batch_anchors):
    NCH = 6
    BS = N // NCH
    h = obj_reg_cls_heads.reshape(L * B, N, 85)
    a = batch_anchors.reshape(L * B, N, 5)
    obj = h[:, :, 0].reshape(L * B, NCH, 1, BS)
    regs = [h[:, :, 1 + i].reshape(L * B, NCH, 1, BS) for i in range(4)]
    ancs = [a[:, :, i].reshape(L * B, NCH, 1, BS) for i in range(5)]

    row_spec = pl.BlockSpec((1, 1, 1, BS), lambda i, j: (i, j, 0, 0))
    outs = pl.pallas_call(
        _decode_kernel,
        grid=(L * B, NCH),
        in_specs=[pl.BlockSpec((1, BS, 85), lambda i, j: (i, j, 0))]
        + [row_spec] * 10,
        out_specs=[row_spec] * 6,
        out_shape=[jax.ShapeDtypeStruct((L * B, NCH, 1, BS), jnp.float32)] * 6,
    )(h, obj, *regs, *ancs)
    def to_img(t):
        return t.reshape(L, B, N).transpose(1, 0, 2).reshape(B, L * N)
    return tuple(to_img(t) for t in outs)


NPAD = 1024


def _nms_kernel(sc_ref, cl_ref, x1_ref, y1_ref, x2_ref, y2_ref,
                os_ref, oc_ref, ob_ref, s_mat):
    sc = sc_ref[0, 0]
    x1 = x1_ref[0, 0]
    y1 = y1_ref[0, 0]
    x2 = x2_ref[0, 0]
    y2 = y2_ref[0, 0]
    areas = jnp.clip((x2 - x1) * (y2 - y1), 0.0001, None)

    colx1 = x1.reshape(1, 8, 128)
    coly1 = y1.reshape(1, 8, 128)
    colx2 = x2.reshape(1, 8, 128)
    coly2 = y2.reshape(1, 8, 128)
    colar = areas.reshape(1, 8, 128)
    colidx = jax.lax.broadcasted_iota(jnp.int32, (1, 8, 128), 1) * 128 + \
        jax.lax.broadcasted_iota(jnp.int32, (1, 8, 128), 2)
    for r in range(NPAD // 128):
        sl = slice(r * 128, (r + 1) * 128)
        rx1 = x1[sl].reshape(128, 1, 1)
        ry1 = y1[sl].reshape(128, 1, 1)
        rx2 = x2[sl].reshape(128, 1, 1)
        ry2 = y2[sl].reshape(128, 1, 1)
        rar = areas[sl].reshape(128, 1, 1)
        ridx = jax.lax.broadcasted_iota(jnp.int32, (128, 1, 1), 0) + r * 128
        sx = jnp.clip(jnp.minimum(rx2, colx2) - jnp.maximum(rx1, colx1), 0.0, None)
        sy = jnp.clip(jnp.minimum(ry2, coly2) - jnp.maximum(ry1, coly1), 0.0, None)
        overlap = sx * sy
        union = jnp.clip(rar + colar - overlap, 0.0001, None)
        iou = overlap / union
        sup = (iou >= NMS_THR) & (colidx > ridx)
        s_mat[sl] = sup.astype(jnp.float32)

    alive0 = (sc > MIN_SCORE).astype(jnp.float32).reshape(8, 128)
    flat = jax.lax.broadcasted_iota(jnp.int32, (8, 128), 0) * 128 + \
        jax.lax.broadcasted_iota(jnp.int32, (8, 128), 1)

    def first_alive(al, lo):
        return jnp.min(jnp.where((al > 0.0) & (flat >= lo), flat, TOPN))

    def cond(state):
        return state[0] < TOPN

    def body(state):
        i, al = state
        al = al * (1.0 - s_mat[i])
        return (first_alive(al, i + 1), al)

    _, kept2 = jax.lax.while_loop(
        cond, body, (first_alive(alive0, 0), alive0))

    x = kept2
    for sh in (1, 2, 4, 8, 16, 32, 64):
        x = x + jnp.pad(x, ((0, 0), (sh, 0)))[:, :128]
    row_tot = x[:, 127:128]
    y = row_tot
    for sh in (1, 2, 4):
        y = y + jnp.pad(y, ((sh, 0), (0, 0)))[:8, :]
    rank2 = x + (y - row_tot) - kept2

    kept = kept2.reshape(NPAD)
    rank = rank2.reshape(NPAD)
    pos = jnp.where((kept > 0.0) & (rank < MAX_OBJ), rank, jnp.float32(MAX_OBJ))
    posi = pos.astype(jnp.int32)
    slot = jax.lax.broadcasted_iota(jnp.int32, (MAX_OBJ, NPAD), 0)
    m = (posi.reshape(1, NPAD) == slot).astype(jnp.float32)
    hit = jnp.sum(m, axis=1)
    out_s = jnp.sum(m * sc.reshape(1, NPAD), axis=1) - (1.0 - hit)
    out_c = jnp.sum(m * cl_ref[0, 0].reshape(1, NPAD), axis=1) - (1.0 - hit)
    ox1 = jnp.sum(m * x1.reshape(1, NPAD), axis=1)
    oy1 = jnp.sum(m * y1.reshape(1, NPAD), axis=1)
    ox2 = jnp.sum(m * x2.reshape(1, NPAD), axis=1)
    oy2 = jnp.sum(m * y2.reshape(1, NPAD), axis=1)
    os_ref[0, 0] = out_s
    oc_ref[0, 0] = out_c
    ob_ref[0] = jnp.stack([ox1, oy1, ox2, oy2], axis=1)


def _nms(sc, cl, x1, y1, x2, y2):
    from jax.experimental.pallas import tpu as pltpu
    B_ = sc.shape[0]
    vec = pl.BlockSpec((1, 1, NPAD), lambda i: (i, 0, 0))
    ovec = pl.BlockSpec((1, 1, MAX_OBJ), lambda i: (i, 0, 0))
    obox = pl.BlockSpec((1, MAX_OBJ, 4), lambda i: (i, 0, 0))
    r3 = lambda t: t.reshape(B_, 1, NPAD)
    return pl.pallas_call(
        _nms_kernel,
        grid=(B_,),
        in_specs=[vec] * 6,
        out_specs=[ovec, ovec, obox],
        out_shape=[
            jax.ShapeDtypeStruct((B_, 1, MAX_OBJ), jnp.float32),
            jax.ShapeDtypeStruct((B_, 1, MAX_OBJ), jnp.float32),
            jax.ShapeDtypeStruct((B_, MAX_OBJ, 4), jnp.float32),
        ],
        scratch_shapes=[pltpu.VMEM((NPAD, 8, 128), jnp.float32)],
    )(r3(sc), r3(cl), r3(x1), r3(y1), r3(x2), r3(y2))


def kernel(obj_reg_cls_heads, batch_anchors):
    s, c, x1, y1, x2, y2 = _decode(obj_reg_cls_heads, batch_anchors)
    masked = jnp.where(s > MIN_SCORE, s, jnp.float32(-1.0))
    topv, topi = jax.lax.top_k(masked, TOPN)

    def pad(t):
        return jnp.pad(t, ((0, 0), (0, NPAD - TOPN)))

    sc = jnp.pad(topv, ((0, 0), (0, NPAD - TOPN)), constant_values=-1.0)
    cl = pad(jnp.take_along_axis(c, topi, axis=1))
    gx1 = pad(jnp.take_along_axis(x1, topi, axis=1))
    gy1 = pad(jnp.take_along_axis(y1, topi, axis=1))
    gx2 = pad(jnp.take_along_axis(x2, topi, axis=1))
    gy2 = pad(jnp.take_along_axis(y2, topi, axis=1))

    out_s, out_c, out_b = _nms(sc, cl, gx1, gy1, gx2, gy2)
    return out_s[:, 0, :], out_c[:, 0, :], out_b


# speedup vs baseline: 1.3193x; 1.3193x over previous
import jax
import jax.numpy as jnp
from jax import lax
from jax.experimental import pallas as pl
from jax.experimental.pallas import tpu as pltpu

TOPN = 1000
MIN_SCORE = 0.05
NMS_THR = 0.5
MAX_OBJ = 100

L = 3
B = 4
N = 17328
NT = 52224
ROWS = NT // 128
NPAD = 1024
CH = 24
KEY_MIN = -2147483648


def _decode_kernel(h_ref, obj_ref, r0_ref, r1_ref, r2_ref, r3_ref,
                   a0_ref, a1_ref, a2_ref, a3_ref, a4_ref,
                   s_ref, c_ref, x1_ref, y1_ref, x2_ref, y2_ref, k_ref):
    h = h_ref[0]
    cls = h[:, 5:85]
    m = jnp.max(cls, axis=1)
    iota = jax.lax.broadcasted_iota(jnp.int32, cls.shape, 1)
    c = jnp.min(jnp.where(cls == m[:, None], iota, 10_000), axis=1)
    obj = obj_ref[0, 0, 0]
    s = m * obj
    masked = jnp.where(s > MIN_SCORE, s, jnp.float32(-1.0))
    bits = lax.bitcast_convert_type(masked, jnp.int32)
    key = jnp.where(bits >= 0, bits,
                    jnp.bitwise_xor(jnp.bitwise_not(bits), jnp.int32(KEY_MIN)))
    a0 = a0_ref[0, 0, 0]; a1 = a1_ref[0, 0, 0]; a2 = a2_ref[0, 0, 0]
    a3 = a3_ref[0, 0, 0]; a4 = a4_ref[0, 0, 0]
    cx = (r0_ref[0, 0, 0] + a0) * a4
    cy = (r1_ref[0, 0, 0] + a1) * a4
    w = r2_ref[0, 0, 0] * a2 * a4
    hh = r3_ref[0, 0, 0] * a3 * a4
    s_ref[0, 0, 0] = masked
    c_ref[0, 0, 0] = c.astype(jnp.float32)
    x1_ref[0, 0, 0] = (cx - w * 0.5).astype(jnp.int32).astype(jnp.float32)
    y1_ref[0, 0, 0] = (cy - hh * 0.5).astype(jnp.int32).astype(jnp.float32)
    x2_ref[0, 0, 0] = (cx + w * 0.5).astype(jnp.int32).astype(jnp.float32)
    y2_ref[0, 0, 0] = (cy + hh * 0.5).astype(jnp.int32).astype(jnp.float32)
    k_ref[0, 0, 0] = key


def _decode(obj_reg_cls_heads, batch_anchors):
    NCH = 6
    BS = N // NCH
    h = obj_reg_cls_heads.reshape(L * B, N, 85)
    a = batch_anchors.reshape(L * B, N, 5)
    obj = h[:, :, 0].reshape(L * B, NCH, 1, BS)
    regs = [h[:, :, 1 + i].reshape(L * B, NCH, 1, BS) for i in range(4)]
    ancs = [a[:, :, i].reshape(L * B, NCH, 1, BS) for i in range(5)]

    row_spec = pl.BlockSpec((1, 1, 1, BS), lambda i, j: (i, j, 0, 0))
    outs = pl.pallas_call(
        _decode_kernel,
        grid=(L * B, NCH),
        in_specs=[pl.BlockSpec((1, BS, 85), lambda i, j: (i, j, 0))]
        + [row_spec] * 10,
        out_specs=[row_spec] * 7,
        out_shape=[jax.ShapeDtypeStruct((L * B, NCH, 1, BS), jnp.float32)] * 6
        + [jax.ShapeDtypeStruct((L * B, NCH, 1, BS), jnp.int32)],
    )(h, obj, *regs, *ancs)

    def to_img(t):
        return t.reshape(L, B, N).transpose(1, 0, 2).reshape(B, L * N)
    return tuple(to_img(t) for t in outs)


def _thresh_kernel(k_ref, v_ref):
    key = k_ref[0, 0]
    uku = lax.bitcast_convert_type(key, jnp.uint32) ^ jnp.uint32(0x80000000)

    def body(i, t):
        b = (31 - i).astype(jnp.uint32)
        cand = t | (jnp.uint32(1) << b)
        cnt = jnp.sum((uku >= cand).astype(jnp.int32))
        return jnp.where(cnt >= TOPN, cand, t)

    t = lax.fori_loop(0, 32, body, jnp.uint32(0))
    vs = lax.bitcast_convert_type(t ^ jnp.uint32(0x80000000), jnp.int32)
    v_ref[0, 0, :] = jnp.zeros((16,), jnp.int32) + vs


def _thresh(keyp):
    return pl.pallas_call(
        _thresh_kernel,
        grid=(B,),
        in_specs=[pl.BlockSpec((1, 1, NT), lambda i: (i, 0, 0))],
        out_specs=pl.BlockSpec((1, 1, 16), lambda i: (i, 0, 0)),
        out_shape=jax.ShapeDtypeStruct((B, 1, 16), jnp.int32),
    )(keyp.reshape(B, 1, NT))


def _ctc_kernel(k_ref, s_ref, c_ref, x1_ref, y1_ref, x2_ref, y2_ref,
                v_ref, o_ref):
    key = k_ref[0]
    V = jnp.max(v_ref[0, 0])
    mh = (key > V).astype(jnp.float32)
    mt = (key == V).astype(jnp.float32)

    def exprefix(m):
        x = m
        for sh in (1, 2, 4, 8, 16, 32, 64):
            x = x + jnp.pad(x, ((0, 0), (sh, 0)))[:, :128]
        rt = x[:, 127:128]
        y = rt
        for sh in (1, 2, 4, 8, 16, 32, 64, 128, 256):
            y = y + jnp.pad(y, ((sh, 0), (0, 0)))[:ROWS, :]
        return x + (y - rt) - m

    eh = exprefix(mh)
    et = exprefix(mt)
    A = jnp.sum(mh)
    tie_need = jnp.float32(TOPN) - A
    sel = mh + mt * (et < tie_need).astype(jnp.float32)
    pos = jnp.where(mh > 0.0, eh, A + et)
    posi = jnp.where(sel > 0.0, pos.astype(jnp.int32), jnp.int32(2 * NPAD))

    s = s_ref[0]
    b1 = s.astype(jnp.bfloat16)
    r1 = s - b1.astype(jnp.float32)
    b2 = r1.astype(jnp.bfloat16)
    b3 = (r1 - b2.astype(jnp.float32)).astype(jnp.bfloat16)
    payloads = [b1.astype(jnp.float32), b2.astype(jnp.float32),
                b3.astype(jnp.float32), c_ref[0], x1_ref[0], y1_ref[0],
                x2_ref[0], y2_ref[0]]

    slot = jax.lax.broadcasted_iota(jnp.int32, (1, NPAD), 1)
    acc = jnp.zeros((8, NPAD), jnp.float32)
    pflats = [p.reshape(NT) for p in payloads]
    posf = posi.reshape(NT)
    CHE = CH * 128
    for ci in range(ROWS // CH):
        rs = slice(ci * CHE, (ci + 1) * CHE)
        lhs = jnp.concatenate(
            [p[rs].reshape(1, CHE) for p in pflats], axis=0
        ).astype(jnp.bfloat16)
        pcol = posf[rs].reshape(CHE, 1)
        oh = (pcol == slot).astype(jnp.bfloat16)
        acc = acc + jax.lax.dot_general(
            lhs, oh, (((1,), (0,)), ((), ())),
            preferred_element_type=jnp.float32)
    o_ref[0] = acc


def _compact(key3, s3, c3, x13, y13, x23, y23, varr):
    blk = pl.BlockSpec((1, ROWS, 128), lambda i: (i, 0, 0))
    return pl.pallas_call(
        _ctc_kernel,
        grid=(B,),
        in_specs=[blk] * 7 + [pl.BlockSpec((1, 1, 16), lambda i: (i, 0, 0))],
        out_specs=pl.BlockSpec((1, 8, NPAD), lambda i: (i, 0, 0)),
        out_shape=jax.ShapeDtypeStruct((B, 8, NPAD), jnp.float32),
    )(key3, s3, c3, x13, y13, x23, y23, varr)


def _nms_kernel(p_ref, os_ref, oc_ref, ob_ref, s_mat):
    flat = jax.lax.broadcasted_iota(jnp.int32, (8, 128), 0) * 128 + \
        jax.lax.broadcasted_iota(jnp.int32, (8, 128), 1)

    sc0 = (p_ref[0, 0] + p_ref[0, 1] + p_ref[0, 2]).reshape(8, 128)
    cl0 = p_ref[0, 3].reshape(8, 128)
    x10 = p_ref[0, 4].reshape(8, 128)
    y10 = p_ref[0, 5].reshape(8, 128)
    x20 = p_ref[0, 6].reshape(8, 128)
    y20 = p_ref[0, 7].reshape(8, 128)

    sflat = sc0.reshape(NPAD)
    fflat = flat.reshape(NPAD)
    scol = sc0.reshape(1, 8, 128)
    fcol = flat.reshape(1, 8, 128)
    rparts = []
    for r in range(8):
        sl = slice(r * 128, (r + 1) * 128)
        srow = sflat[sl].reshape(128, 1, 1)
        frow = fflat[sl].reshape(128, 1, 1)
        gt = (scol > srow).astype(jnp.float32)
        eq = ((scol == srow) & (fcol < frow)).astype(jnp.float32)
        rparts.append(jnp.sum(jnp.sum(gt + eq, axis=2), axis=1))
    ranki = jnp.concatenate(rparts).astype(jnp.int32)
    rcol = ranki.reshape(1, 8, 128)

    payloads = [sflat, cl0.reshape(NPAD), x10.reshape(NPAD),
                y10.reshape(NPAD), x20.reshape(NPAD), y20.reshape(NPAD)]
    pcols = [p.reshape(1, 8, 128) for p in payloads]
    parts = [[] for _ in payloads]
    for r in range(8):
        ridx = jax.lax.broadcasted_iota(jnp.int32, (128, 1, 1), 0) + r * 128
        mc = (rcol == ridx).astype(jnp.float32)
        for pi, pc in enumerate(pcols):
            parts[pi].append(jnp.sum(jnp.sum(mc * pc, axis=2), axis=1))
    sc, cl, x1, y1, x2, y2 = [jnp.concatenate(ps) for ps in parts]

    areas = jnp.clip((x2 - x1) * (y2 - y1), 0.0001, None)
    colx1 = x1.reshape(1, 8, 128)
    coly1 = y1.reshape(1, 8, 128)
    colx2 = x2.reshape(1, 8, 128)
    coly2 = y2.reshape(1, 8, 128)
    colar = areas.reshape(1, 8, 128)
    colidx = fcol
    for r in range(NPAD // 128):
        sl = slice(r * 128, (r + 1) * 128)
        rx1 = x1[sl].reshape(128, 1, 1)
        ry1 = y1[sl].reshape(128, 1, 1)
        rx2 = x2[sl].reshape(128, 1, 1)
        ry2 = y2[sl].reshape(128, 1, 1)
        rar = areas[sl].reshape(128, 1, 1)
        ridx = jax.lax.broadcasted_iota(jnp.int32, (128, 1, 1), 0) + r * 128
        sx = jnp.clip(jnp.minimum(rx2, colx2) - jnp.maximum(rx1, colx1), 0.0, None)
        sy = jnp.clip(jnp.minimum(ry2, coly2) - jnp.maximum(ry1, coly1), 0.0, None)
        overlap = sx * sy
        union = jnp.clip(rar + colar - overlap, 0.0001, None)
        iou = overlap / union
        sup = (iou >= NMS_THR) & (colidx > ridx)
        s_mat[sl] = sup.astype(jnp.float32)

    alive0 = (sc > MIN_SCORE).astype(jnp.float32).reshape(8, 128)

    def body(i, al):
        row = s_mat[i]
        ai = jnp.max(jnp.where(flat == i, al, 0.0))
        return al * (1.0 - ai * row)

    kept2 = jax.lax.fori_loop(0, TOPN, body, alive0)

    x = kept2
    for sh in (1, 2, 4, 8, 16, 32, 64):
        x = x + jnp.pad(x, ((0, 0), (sh, 0)))[:, :128]
    row_tot = x[:, 127:128]
    y = row_tot
    for sh in (1, 2, 4):
        y = y + jnp.pad(y, ((sh, 0), (0, 0)))[:8, :]
    rank2 = x + (y - row_tot) - kept2

    kept = kept2.reshape(NPAD)
    rank = rank2.reshape(NPAD)
    pos = jnp.where((kept > 0.0) & (rank < MAX_OBJ), rank, jnp.float32(MAX_OBJ))
    posi = pos.astype(jnp.int32)
    slot = jax.lax.broadcasted_iota(jnp.int32, (MAX_OBJ, NPAD), 0)
    m = (posi.reshape(1, NPAD) == slot).astype(jnp.float32)
    hit = jnp.sum(m, axis=1)
    out_s = jnp.sum(m * sc.reshape(1, NPAD), axis=1) - (1.0 - hit)
    out_c = jnp.sum(m * cl.reshape(1, NPAD), axis=1) - (1.0 - hit)
    ox1 = jnp.sum(m * x1.reshape(1, NPAD), axis=1)
    oy1 = jnp.sum(m * y1.reshape(1, NPAD), axis=1)
    ox2 = jnp.sum(m * x2.reshape(1, NPAD), axis=1)
    oy2 = jnp.sum(m * y2.reshape(1, NPAD), axis=1)
    os_ref[0, 0] = out_s
    oc_ref[0, 0] = out_c
    ob_ref[0] = jnp.stack([ox1, oy1, ox2, oy2], axis=1)


def _nms(packed):
    ovec = pl.BlockSpec((1, 1, MAX_OBJ), lambda i: (i, 0, 0))
    obox = pl.BlockSpec((1, MAX_OBJ, 4), lambda i: (i, 0, 0))
    return pl.pallas_call(
        _nms_kernel,
        grid=(B,),
        in_specs=[pl.BlockSpec((1, 8, NPAD), lambda i: (i, 0, 0))],
        out_specs=[ovec, ovec, obox],
        out_shape=[
            jax.ShapeDtypeStruct((B, 1, MAX_OBJ), jnp.float32),
            jax.ShapeDtypeStruct((B, 1, MAX_OBJ), jnp.float32),
            jax.ShapeDtypeStruct((B, MAX_OBJ, 4), jnp.float32),
        ],
        scratch_shapes=[pltpu.VMEM((NPAD, 8, 128), jnp.float32)],
    )(packed)


def kernel(obj_reg_cls_heads, batch_anchors):
    s, c, x1, y1, x2, y2, key = _decode(obj_reg_cls_heads, batch_anchors)
    padf = lambda t: jnp.pad(t, ((0, 0), (0, NT - L * N))).reshape(B, ROWS, 128)
    keyp = jnp.pad(key, ((0, 0), (0, NT - L * N)),
                   constant_values=KEY_MIN)
    varr = _thresh(keyp)
    packed = _compact(keyp.reshape(B, ROWS, 128), padf(s), padf(c),
                      padf(x1), padf(y1), padf(x2), padf(y2), varr)
    out_s, out_c, out_b = _nms(packed)
    return out_s[:, 0, :], out_c[:, 0, :], out_b


# speedup vs baseline: 1.3764x; 1.0433x over previous
import jax
import jax.numpy as jnp
from jax import lax
from jax.experimental import pallas as pl
from jax.experimental.pallas import tpu as pltpu

TOPN = 1000
MIN_SCORE = 0.05
NMS_THR = 0.5
MAX_OBJ = 100

L = 3
B = 4
N = 17328
NT = 52224
ROWS = NT // 128
NPAD = 1024
CH = 24
KEY_MIN = -2147483648


def _decode_kernel(h_ref, obj_ref, r0_ref, r1_ref, r2_ref, r3_ref,
                   a0_ref, a1_ref, a2_ref, a3_ref, a4_ref,
                   s_ref, c_ref, x1_ref, y1_ref, x2_ref, y2_ref, k_ref):
    h = h_ref[0]
    cls = h[:, 5:85]
    m = jnp.max(cls, axis=1)
    iota = jax.lax.broadcasted_iota(jnp.int32, cls.shape, 1)
    c = jnp.min(jnp.where(cls == m[:, None], iota, 10_000), axis=1)
    obj = obj_ref[0, 0, 0]
    s = m * obj
    masked = jnp.where(s > MIN_SCORE, s, jnp.float32(-1.0))
    bits = lax.bitcast_convert_type(masked, jnp.int32)
    key = jnp.where(bits >= 0, bits,
                    jnp.bitwise_xor(jnp.bitwise_not(bits), jnp.int32(KEY_MIN)))
    a0 = a0_ref[0, 0, 0]; a1 = a1_ref[0, 0, 0]; a2 = a2_ref[0, 0, 0]
    a3 = a3_ref[0, 0, 0]; a4 = a4_ref[0, 0, 0]
    cx = (r0_ref[0, 0, 0] + a0) * a4
    cy = (r1_ref[0, 0, 0] + a1) * a4
    w = r2_ref[0, 0, 0] * a2 * a4
    hh = r3_ref[0, 0, 0] * a3 * a4
    s_ref[0, 0, 0] = masked
    c_ref[0, 0, 0] = c.astype(jnp.float32)
    x1_ref[0, 0, 0] = (cx - w * 0.5).astype(jnp.int32).astype(jnp.float32)
    y1_ref[0, 0, 0] = (cy - hh * 0.5).astype(jnp.int32).astype(jnp.float32)
    x2_ref[0, 0, 0] = (cx + w * 0.5).astype(jnp.int32).astype(jnp.float32)
    y2_ref[0, 0, 0] = (cy + hh * 0.5).astype(jnp.int32).astype(jnp.float32)
    k_ref[0, 0, 0] = key


def _decode(obj_reg_cls_heads, batch_anchors):
    NCH = 6
    BS = N // NCH
    h = obj_reg_cls_heads.reshape(L * B, N, 85)
    a = batch_anchors.reshape(L * B, N, 5)
    obj = h[:, :, 0].reshape(L * B, NCH, 1, BS)
    regs = [h[:, :, 1 + i].reshape(L * B, NCH, 1, BS) for i in range(4)]
    ancs = [a[:, :, i].reshape(L * B, NCH, 1, BS) for i in range(5)]

    in_row = pl.BlockSpec((1, 1, 1, BS),
                          lambda i, j: ((i % L) * B + (i // L), j, 0, 0))
    out_row = pl.BlockSpec((1, 1, 1, BS), lambda i, j: (i, j, 0, 0))
    outs = pl.pallas_call(
        _decode_kernel,
        grid=(B * L, NCH),
        in_specs=[pl.BlockSpec((1, BS, 85),
                               lambda i, j: ((i % L) * B + (i // L), j, 0))]
        + [in_row] * 10,
        out_specs=[out_row] * 7,
        out_shape=[jax.ShapeDtypeStruct((B * L, NCH, 1, BS), jnp.float32)] * 6
        + [jax.ShapeDtypeStruct((B * L, NCH, 1, BS), jnp.int32)],
    )(h, obj, *regs, *ancs)

    return tuple(t.reshape(B, L * N) for t in outs)


def _thresh_kernel(k_ref, v_ref):
    key = k_ref[0]
    uku = lax.bitcast_convert_type(key, jnp.uint32) ^ jnp.uint32(0x80000000)

    def body(i, t):
        b = (31 - i).astype(jnp.uint32)
        cand = t | (jnp.uint32(1) << b)
        cnt = jnp.sum((uku >= cand).astype(jnp.int32))
        return jnp.where(cnt >= TOPN, cand, t)

    t = lax.fori_loop(0, 32, body, jnp.uint32(0))
    vs = lax.bitcast_convert_type(t ^ jnp.uint32(0x80000000), jnp.int32)
    v_ref[0, 0, :] = jnp.zeros((16,), jnp.int32) + vs


def _thresh(key3):
    return pl.pallas_call(
        _thresh_kernel,
        grid=(B,),
        in_specs=[pl.BlockSpec((1, ROWS, 128), lambda i: (i, 0, 0))],
        out_specs=pl.BlockSpec((1, 1, 16), lambda i: (i, 0, 0)),
        out_shape=jax.ShapeDtypeStruct((B, 1, 16), jnp.int32),
    )(key3)


def _ctc_kernel(k_ref, s_ref, c_ref, x1_ref, y1_ref, x2_ref, y2_ref,
                v_ref, o_ref):
    key = k_ref[0]
    V = jnp.max(v_ref[0, 0])
    mh = (key > V).astype(jnp.float32)
    mt = (key == V).astype(jnp.float32)

    def exprefix(m):
        x = m
        for sh in (1, 2, 4, 8, 16, 32, 64):
            x = x + jnp.pad(x, ((0, 0), (sh, 0)))[:, :128]
        rt = x[:, 127:128]
        y = rt
        for sh in (1, 2, 4, 8, 16, 32, 64, 128, 256):
            y = y + jnp.pad(y, ((sh, 0), (0, 0)))[:ROWS, :]
        return x + (y - rt) - m

    eh = exprefix(mh)
    et = exprefix(mt)
    A = jnp.sum(mh)
    tie_need = jnp.float32(TOPN) - A
    sel = mh + mt * (et < tie_need).astype(jnp.float32)
    pos = jnp.where(mh > 0.0, eh, A + et)
    posi = jnp.where(sel > 0.0, pos.astype(jnp.int32), jnp.int32(2 * NPAD))

    s = s_ref[0]
    b1 = s.astype(jnp.bfloat16)
    r1 = s - b1.astype(jnp.float32)
    b2 = r1.astype(jnp.bfloat16)
    b3 = (r1 - b2.astype(jnp.float32)).astype(jnp.bfloat16)
    payloads = [b1.astype(jnp.float32), b2.astype(jnp.float32),
                b3.astype(jnp.float32), c_ref[0], x1_ref[0], y1_ref[0],
                x2_ref[0], y2_ref[0]]

    slot = jax.lax.broadcasted_iota(jnp.int32, (1, NPAD), 1)
    acc = jnp.zeros((8, NPAD), jnp.float32)
    pflats = [p.reshape(NT) for p in payloads]
    posf = posi.reshape(NT)
    CHE = CH * 128
    for ci in range(ROWS // CH):
        rs = slice(ci * CHE, (ci + 1) * CHE)
        lhs = jnp.concatenate(
            [p[rs].reshape(1, CHE) for p in pflats], axis=0
        ).astype(jnp.bfloat16)
        pcol = posf[rs].reshape(CHE, 1)
        oh = (pcol == slot).astype(jnp.bfloat16)
        acc = acc + jax.lax.dot_general(
            lhs, oh, (((1,), (0,)), ((), ())),
            preferred_element_type=jnp.float32)
    o_ref[0] = acc


def _compact(key3, s3, c3, x13, y13, x23, y23, varr):
    blk = pl.BlockSpec((1, ROWS, 128), lambda i: (i, 0, 0))
    return pl.pallas_call(
        _ctc_kernel,
        grid=(B,),
        in_specs=[blk] * 7 + [pl.BlockSpec((1, 1, 16), lambda i: (i, 0, 0))],
        out_specs=pl.BlockSpec((1, 8, NPAD), lambda i: (i, 0, 0)),
        out_shape=jax.ShapeDtypeStruct((B, 8, NPAD), jnp.float32),
    )(key3, s3, c3, x13, y13, x23, y23, varr)


def _nms_kernel(p_ref, os_ref, oc_ref, ob_ref, s_mat):
    flat = jax.lax.broadcasted_iota(jnp.int32, (8, 128), 0) * 128 + \
        jax.lax.broadcasted_iota(jnp.int32, (8, 128), 1)

    sc0 = (p_ref[0, 0] + p_ref[0, 1] + p_ref[0, 2]).reshape(8, 128)
    cl0 = p_ref[0, 3].reshape(8, 128)
    x10 = p_ref[0, 4].reshape(8, 128)
    y10 = p_ref[0, 5].reshape(8, 128)
    x20 = p_ref[0, 6].reshape(8, 128)
    y20 = p_ref[0, 7].reshape(8, 128)

    sflat = sc0.reshape(NPAD)
    fflat = flat.reshape(NPAD)
    scol = sc0.reshape(1, 8, 128)
    fcol = flat.reshape(1, 8, 128)
    rparts = []
    for r in range(8):
        sl = slice(r * 128, (r + 1) * 128)
        srow = sflat[sl].reshape(128, 1, 1)
        frow = fflat[sl].reshape(128, 1, 1)
        gt = (scol > srow).astype(jnp.float32)
        eq = ((scol == srow) & (fcol < frow)).astype(jnp.float32)
        rparts.append(jnp.sum(jnp.sum(gt + eq, axis=2), axis=1))
    ranki = jnp.concatenate(rparts).astype(jnp.int32)
    rcol = ranki.reshape(1, 8, 128)

    payloads = [sflat, cl0.reshape(NPAD), x10.reshape(NPAD),
                y10.reshape(NPAD), x20.reshape(NPAD), y20.reshape(NPAD)]
    pcols = [p.reshape(1, 8, 128) for p in payloads]
    parts = [[] for _ in payloads]
    for r in range(8):
        ridx = jax.lax.broadcasted_iota(jnp.int32, (128, 1, 1), 0) + r * 128
        mc = (rcol == ridx).astype(jnp.float32)
        for pi, pc in enumerate(pcols):
            parts[pi].append(jnp.sum(jnp.sum(mc * pc, axis=2), axis=1))
    sc, cl, x1, y1, x2, y2 = [jnp.concatenate(ps) for ps in parts]

    areas = jnp.clip((x2 - x1) * (y2 - y1), 0.0001, None)
    colx1 = x1.reshape(1, 8, 128)
    coly1 = y1.reshape(1, 8, 128)
    colx2 = x2.reshape(1, 8, 128)
    coly2 = y2.reshape(1, 8, 128)
    colar = areas.reshape(1, 8, 128)
    colidx = fcol
    for r in range(NPAD // 128):
        sl = slice(r * 128, (r + 1) * 128)
        rx1 = x1[sl].reshape(128, 1, 1)
        ry1 = y1[sl].reshape(128, 1, 1)
        rx2 = x2[sl].reshape(128, 1, 1)
        ry2 = y2[sl].reshape(128, 1, 1)
        rar = areas[sl].reshape(128, 1, 1)
        ridx = jax.lax.broadcasted_iota(jnp.int32, (128, 1, 1), 0) + r * 128
        sx = jnp.clip(jnp.minimum(rx2, colx2) - jnp.maximum(rx1, colx1), 0.0, None)
        sy = jnp.clip(jnp.minimum(ry2, coly2) - jnp.maximum(ry1, coly1), 0.0, None)
        overlap = sx * sy
        union = jnp.clip(rar + colar - overlap, 0.0001, None)
        iou = overlap / union
        sup = (iou >= NMS_THR) & (colidx > ridx)
        s_mat[sl] = sup.astype(jnp.float32)

    alive0 = (sc > MIN_SCORE).astype(jnp.float32).reshape(8, 128)

    def body(i, al):
        row = s_mat[i]
        ai = jnp.max(jnp.where(flat == i, al, 0.0))
        return al * (1.0 - ai * row)

    kept2 = jax.lax.fori_loop(0, TOPN, body, alive0)

    x = kept2
    for sh in (1, 2, 4, 8, 16, 32, 64):
        x = x + jnp.pad(x, ((0, 0), (sh, 0)))[:, :128]
    row_tot = x[:, 127:128]
    y = row_tot
    for sh in (1, 2, 4):
        y = y + jnp.pad(y, ((sh, 0), (0, 0)))[:8, :]
    rank2 = x + (y - row_tot) - kept2

    kept = kept2.reshape(NPAD)
    rank = rank2.reshape(NPAD)
    pos = jnp.where((kept > 0.0) & (rank < MAX_OBJ), rank, jnp.float32(MAX_OBJ))
    posi = pos.astype(jnp.int32)
    slot = jax.lax.broadcasted_iota(jnp.int32, (MAX_OBJ, NPAD), 0)
    m = (posi.reshape(1, NPAD) == slot).astype(jnp.float32)
    hit = jnp.sum(m, axis=1)
    out_s = jnp.sum(m * sc.reshape(1, NPAD), axis=1) - (1.0 - hit)
    out_c = jnp.sum(m * cl.reshape(1, NPAD), axis=1) - (1.0 - hit)
    ox1 = jnp.sum(m * x1.reshape(1, NPAD), axis=1)
    oy1 = jnp.sum(m * y1.reshape(1, NPAD), axis=1)
    ox2 = jnp.sum(m * x2.reshape(1, NPAD), axis=1)
    oy2 = jnp.sum(m * y2.reshape(1, NPAD), axis=1)
    os_ref[0, 0] = out_s
    oc_ref[0, 0] = out_c
    ob_ref[0] = jnp.stack([ox1, oy1, ox2, oy2], axis=1)


def _nms(packed):
    ovec = pl.BlockSpec((1, 1, MAX_OBJ), lambda i: (i, 0, 0))
    obox = pl.BlockSpec((1, MAX_OBJ, 4), lambda i: (i, 0, 0))
    return pl.pallas_call(
        _nms_kernel,
        grid=(B,),
        in_specs=[pl.BlockSpec((1, 8, NPAD), lambda i: (i, 0, 0))],
        out_specs=[ovec, ovec, obox],
        out_shape=[
            jax.ShapeDtypeStruct((B, 1, MAX_OBJ), jnp.float32),
            jax.ShapeDtypeStruct((B, 1, MAX_OBJ), jnp.float32),
            jax.ShapeDtypeStruct((B, MAX_OBJ, 4), jnp.float32),
        ],
        scratch_shapes=[pltpu.VMEM((NPAD, 8, 128), jnp.float32)],
    )(packed)


def kernel(obj_reg_cls_heads, batch_anchors):
    s, c, x1, y1, x2, y2, key = _decode(obj_reg_cls_heads, batch_anchors)
    padf = lambda t: jnp.pad(t, ((0, 0), (0, NT - L * N))).reshape(B, ROWS, 128)
    key3 = jnp.pad(key, ((0, 0), (0, NT - L * N)),
                   constant_values=KEY_MIN).reshape(B, ROWS, 128)
    varr = _thresh(key3)
    packed = _compact(key3, padf(s), padf(c),
                      padf(x1), padf(y1), padf(x2), padf(y2), varr)
    out_s, out_c, out_b = _nms(packed)
    return out_s[:, 0, :], out_c[:, 0, :], out_b
